# baseline (speedup 1.0000x reference)
import functools

import jax
import jax.numpy as jnp
from jax import lax
from jax.experimental import pallas as pl
from jax.experimental.pallas import tpu as pltpu
from jax.experimental.pallas import tpu_sc as plsc

N = 10000
E = 320000
H = 128
B = 125
NSUB = 16
NCORE = 2
HH = H // 2
EPW = E // (NCORE * NSUB)
ROWS = E // B // NSUB
NPAD = 10240
NPT = NPAD // NSUB
RB = 2000
B2 = 25
ROWS2 = EPW // B2

_mesh = plsc.VectorSubcoreMesh(core_axis_name="c", subcore_axis_name="s",
                               num_cores=NCORE, num_subcores=NSUB)


@functools.partial(
    pl.kernel,
    out_type=jax.ShapeDtypeStruct((NCORE * NSUB, NPAD), jnp.float32),
    mesh=_mesh,
    scratch_types=[
        pltpu.VMEM((EPW,), jnp.int32),
        pltpu.VMEM((NPAD,), jnp.float32),
    ],
    compiler_params=pltpu.CompilerParams(
        needs_layout_passes=False, use_tc_tiling_on_sc=False),
)
def _deg_count(dst_hbm, out_hbm, dstb, degp):
    c = lax.axis_index("c")
    s = lax.axis_index("s")
    w = c * NSUB + s
    pltpu.sync_copy(dst_hbm.at[pl.ds(w * EPW, EPW)], dstb)
    zeros16 = jnp.zeros((16,), jnp.float32)

    def zbody(k, carry):
        degp[pl.ds(k * 16, 16)] = zeros16
        return carry

    lax.fori_loop(0, NPAD // 16, zbody, 0)
    ones16 = jnp.ones((16,), jnp.float32)

    def cbody(k, carry):
        idx = dstb[pl.ds(k * 16, 16)]
        plsc.addupdate_scatter(degp, [idx], ones16)
        return carry

    lax.fori_loop(0, EPW // 16, cbody, 0)
    pltpu.sync_copy(degp, out_hbm.at[w])


@functools.partial(
    pl.kernel,
    out_type=[jax.ShapeDtypeStruct((NPAD, H), jnp.float32),
              jax.ShapeDtypeStruct((NPAD, H), jnp.float32)],
    mesh=_mesh,
    scratch_types=[
        pltpu.VMEM((ROWS2, B2), jnp.int32),
        pltpu.VMEM((ROWS2, B2), jnp.int32),
        pltpu.VMEM((B2, H), jnp.float32),
        pltpu.VMEM((B2, H), jnp.float32),
        pltpu.VMEM_SHARED((NPAD, H), jnp.float32),
        pltpu.SemaphoreType.DMA,
        pltpu.SemaphoreType.DMA,
    ],
    compiler_params=pltpu.CompilerParams(use_tc_tiling_on_sc=False),
)
def _edge_scatter(y_hbm, src_hbm, dst_hbm, zero_hbm, out0_hbm, out1_hbm,
                  srcb, dstb, g0, g1, acc, sem0, sem1):
    c = lax.axis_index("c")
    s = lax.axis_index("s")
    w = c * NSUB + s
    pltpu.sync_copy(src_hbm.at[pl.ds(w * ROWS2, ROWS2)], srcb)
    pltpu.sync_copy(dst_hbm.at[pl.ds(w * ROWS2, ROWS2)], dstb)
    pltpu.sync_copy(zero_hbm.at[pl.ds(s * NPT, NPT)],
                    acc.at[pl.ds(s * NPT, NPT)])
    plsc.subcore_barrier()

    def start(j, buf, sem):
        pltpu.async_copy(y_hbm.at[srcb.at[j]], buf, sem)

    def wait(buf, sem):
        pltpu.make_async_copy(y_hbm.at[srcb.at[0]], buf, sem).wait()

    def scat(j, buf):
        pltpu.sync_copy(buf, acc.at[dstb.at[j]], add=True)

    start(0, g0, sem0)

    def body(i, carry):
        j = 2 * i
        start(j + 1, g1, sem1)
        wait(g0, sem0)
        scat(j, g0)

        @pl.when(j + 2 < ROWS2)
        def _():
            start(j + 2, g0, sem0)

        wait(g1, sem1)
        scat(j + 1, g1)
        return carry

    lax.fori_loop(0, ROWS2 // 2, body, 0)
    plsc.subcore_barrier()

    @pl.when(c == 0)
    def _():
        pltpu.sync_copy(acc.at[pl.ds(s * NPT, NPT)],
                        out0_hbm.at[pl.ds(s * NPT, NPT)])

    @pl.when(c == 1)
    def _():
        pltpu.sync_copy(acc.at[pl.ds(s * NPT, NPT)],
                        out1_hbm.at[pl.ds(s * NPT, NPT)])


def _dis_body(degp_ref, dis_ref):
    deg = jnp.sum(degp_ref[...], axis=0, keepdims=True) + 1.0
    dis_ref[...] = lax.rsqrt(deg)


def _pre_body(x_ref, wi_ref, bi_ref, w0_ref, dis_ref, h_ref, y_ref):
    h = jnp.maximum(
        jnp.dot(x_ref[...], wi_ref[...], preferred_element_type=jnp.float32)
        + bi_ref[...], 0.0)
    h_ref[...] = h
    y_ref[...] = dis_ref[...] * jnp.dot(
        h, w0_ref[...], preferred_element_type=jnp.float32)


def _ln(x, g, b):
    m = jnp.mean(x, axis=-1, keepdims=True)
    v = jnp.mean((x - m) ** 2, axis=-1, keepdims=True)
    return (x - m) * lax.rsqrt(v + 1e-5) * g + b


def _layer_body(z0_ref, z1_ref, y_ref, dis_ref, h_ref, b_ref, g_ref,
                bb_ref, wn_ref, hn_ref, yn_ref):
    dis = dis_ref[...]
    agg = dis * (z0_ref[...] + z1_ref[...] + y_ref[...]) + b_ref[...]
    hn = jnp.maximum(_ln(agg, g_ref[...], bb_ref[...]), 0.0) + h_ref[...]
    hn_ref[...] = hn
    yn_ref[...] = dis * jnp.dot(
        hn, wn_ref[...], preferred_element_type=jnp.float32)


def _layer_final_body(z0_ref, z1_ref, y_ref, dis_ref, h_ref, b_ref, g_ref,
                      bb_ref, hn_ref):
    agg = dis_ref[...] * (z0_ref[...] + z1_ref[...] + y_ref[...]) + b_ref[...]
    hn_ref[...] = jnp.maximum(_ln(agg, g_ref[...], bb_ref[...]), 0.0) + h_ref[...]


def _heads_body(h_ref, nw1_ref, nb1_ref, ng_ref, nbb_ref, nw2_ref, nb2_ref,
                nw3_ref, nb3_ref, gw1_ref, gb1_ref, gg_ref, gbb_ref, gw2_ref,
                gb2_ref, gw3_ref, gb3_ref, nlog_ref, glog_ref, sacc):
    i = pl.program_id(0)
    blk = h_ref[...]

    @pl.when(i == 0)
    def _():
        sacc[0:1, :] = jnp.zeros((1, H), jnp.float32)
        sacc[1:2, :] = jnp.full((1, H), -jnp.inf, jnp.float32)

    sacc[0:1, :] += jnp.sum(blk, axis=0, keepdims=True)
    sacc[1:2, :] = jnp.maximum(sacc[1:2, :], jnp.max(blk, axis=0, keepdims=True))

    t = jnp.dot(blk, nw1_ref[...], preferred_element_type=jnp.float32) + nb1_ref[...]
    t = jnp.maximum(_ln(t, ng_ref[...], nbb_ref[...]), 0.0)
    t = jnp.maximum(
        jnp.dot(t, nw2_ref[...], preferred_element_type=jnp.float32) + nb2_ref[...],
        0.0)
    nlog_ref[...] = jnp.dot(
        t, nw3_ref[...], preferred_element_type=jnp.float32) + nb3_ref[...]

    @pl.when(i == pl.num_programs(0) - 1)
    def _():
        gm = sacc[0:1, :] * (1.0 / N)
        gx = sacc[1:2, :]
        gh = (jnp.dot(gm, gw1_ref[0:H, :], preferred_element_type=jnp.float32)
              + jnp.dot(gx, gw1_ref[H:2 * H, :], preferred_element_type=jnp.float32)
              + gb1_ref[...])
        gh = jnp.maximum(_ln(gh, gg_ref[...], gbb_ref[...]), 0.0)
        gh = jnp.maximum(
            jnp.dot(gh, gw2_ref[...], preferred_element_type=jnp.float32)
            + gb2_ref[...], 0.0)
        gl = jnp.dot(gh, gw3_ref[...], preferred_element_type=jnp.float32) + gb3_ref[...]
        glog_ref[...] = jnp.broadcast_to(gl, (8, H))


def _row_spec(rows):
    return pl.BlockSpec((rows, H), lambda i: (i, 0))


def _full(shape):
    return pl.BlockSpec(shape, lambda i: tuple(0 for _ in shape))


_GRID = N // RB


def _pre_call(x, wi, bi, w0, dis):
    return pl.pallas_call(
        _pre_body,
        grid=(_GRID,),
        in_specs=[_row_spec(RB), _full((H, H)), _full((1, H)), _full((H, H)),
                  pl.BlockSpec((RB, 1), lambda i: (i, 0))],
        out_specs=[_row_spec(RB), _row_spec(RB)],
        out_shape=[jax.ShapeDtypeStruct((N, H), jnp.float32),
                   jax.ShapeDtypeStruct((N, H), jnp.float32)],
    )(x, wi, bi, w0, dis)


def _dis_call(degp):
    return pl.pallas_call(
        _dis_body,
        out_shape=jax.ShapeDtypeStruct((1, NPAD), jnp.float32),
    )(degp)


def _layer_call(z0, z1, y, dis, h, b, g, bb, wn):
    return pl.pallas_call(
        _layer_body,
        grid=(_GRID,),
        in_specs=[_row_spec(RB), _row_spec(RB), _row_spec(RB),
                  pl.BlockSpec((RB, 1), lambda i: (i, 0)), _row_spec(RB),
                  _full((1, H)), _full((1, H)), _full((1, H)), _full((H, H))],
        out_specs=[_row_spec(RB), _row_spec(RB)],
        out_shape=[jax.ShapeDtypeStruct((N, H), jnp.float32),
                   jax.ShapeDtypeStruct((N, H), jnp.float32)],
    )(z0, z1, y, dis, h, b, g, bb, wn)


def _layer_final_call(z0, z1, y, dis, h, b, g, bb):
    return pl.pallas_call(
        _layer_final_body,
        grid=(_GRID,),
        in_specs=[_row_spec(RB), _row_spec(RB), _row_spec(RB),
                  pl.BlockSpec((RB, 1), lambda i: (i, 0)), _row_spec(RB),
                  _full((1, H)), _full((1, H)), _full((1, H))],
        out_specs=_row_spec(RB),
        out_shape=jax.ShapeDtypeStruct((N, H), jnp.float32),
    )(z0, z1, y, dis, h, b, g, bb)


def _heads_call(h, nw1, nb1, ng, nbb, nw2, nb2, nw3p, nb3p,
                gw1, gb1, gg, gbb, gw2, gb2, gw3p, gb3p):
    return pl.pallas_call(
        _heads_body,
        grid=(_GRID,),
        in_specs=[_row_spec(RB), _full((H, H)), _full((1, H)), _full((1, H)),
                  _full((1, H)), _full((H, H // 2)), _full((1, H // 2)),
                  _full((H // 2, H)), _full((1, H)), _full((2 * H, H)),
                  _full((1, H)), _full((1, H)), _full((1, H)),
                  _full((H, H // 2)), _full((1, H // 2)), _full((H // 2, H)),
                  _full((1, H))],
        out_specs=[_row_spec(RB), pl.BlockSpec((8, H), lambda i: (0, 0))],
        out_shape=[jax.ShapeDtypeStruct((N, H), jnp.float32),
                   jax.ShapeDtypeStruct((8, H), jnp.float32)],
        scratch_shapes=[pltpu.VMEM((8, H), jnp.float32)],
    )(h, nw1, nb1, ng, nbb, nw2, nb2, nw3p, nb3p,
      gw1, gb1, gg, gbb, gw2, gb2, gw3p, gb3p)


def kernel(x, edge_index, W_in, b_in, conv_W, conv_b, ln_g, ln_b, gW1, gb1,
           gln_g, gln_b, gW2, gb2, gW3, gb3, nW1, nb1, nln_g, nln_b, nW2, nb2,
           nW3, nb3):
    src2d = edge_index[0].reshape(E // B2, B2)
    dst2d = edge_index[1].reshape(E // B2, B2)

    degp = _deg_count(edge_index[1])
    dis = _dis_call(degp).reshape(NPAD, 1)[:N]

    zeros = jnp.zeros((NPAD, H), jnp.float32)
    h, y = _pre_call(x, W_in, b_in.reshape(1, H), conv_W[0], dis)
    for i in range(4):
        z0, z1 = _edge_scatter(y, src2d, dst2d, zeros)
        b = conv_b[i].reshape(1, H)
        g = ln_g[i].reshape(1, H)
        bb = ln_b[i].reshape(1, H)
        if i < 3:
            h, y = _layer_call(z0, z1, y, dis, h, b, g, bb, conv_W[i + 1])
        else:
            h = _layer_final_call(z0, z1, y, dis, h, b, g, bb)

    nw3p = jnp.pad(nW3, ((0, 0), (0, H - 2)))
    nb3p = jnp.pad(nb3, (0, H - 2)).reshape(1, H)
    gw3p = jnp.pad(gW3, ((0, 0), (0, H - 2)))
    gb3p = jnp.pad(gb3, (0, H - 2)).reshape(1, H)
    nlog, glog = _heads_call(
        h, nW1, nb1.reshape(1, H), nln_g.reshape(1, H), nln_b.reshape(1, H),
        nW2, nb2.reshape(1, H // 2), nw3p, nb3p,
        gW1, gb1.reshape(1, H), gln_g.reshape(1, H), gln_b.reshape(1, H),
        gW2, gb2.reshape(1, H // 2), gw3p, gb3p)
    return (glog[0:1, 0:2], nlog[:, 0:2])


# speedup vs baseline: 1.8436x; 1.8436x over previous
import functools

import jax
import jax.numpy as jnp
from jax import lax
from jax.experimental import pallas as pl
from jax.experimental.pallas import tpu as pltpu
from jax.experimental.pallas import tpu_sc as plsc

N = 10000
E = 320000
H = 128
B = 250
NSUB = 16
NCORE = 2
HH = H // 2
EPW = E // (NCORE * NSUB)
ROWS = E // B // NSUB
NPAD = 10240
NPT = NPAD // NSUB
RB = 2000
PH = 2
PCH = ROWS // PH

_mesh = plsc.VectorSubcoreMesh(core_axis_name="c", subcore_axis_name="s",
                               num_cores=NCORE, num_subcores=NSUB)


@functools.partial(
    pl.kernel,
    out_type=jax.ShapeDtypeStruct((NCORE * NSUB, NPAD), jnp.float32),
    mesh=_mesh,
    scratch_types=[
        pltpu.VMEM((EPW,), jnp.int32),
        pltpu.VMEM((NPAD,), jnp.float32),
    ],
    compiler_params=pltpu.CompilerParams(
        needs_layout_passes=False, use_tc_tiling_on_sc=False),
)
def _deg_count(dst_hbm, out_hbm, dstb, degp):
    c = lax.axis_index("c")
    s = lax.axis_index("s")
    w = c * NSUB + s
    pltpu.sync_copy(dst_hbm.at[pl.ds(w * EPW, EPW)], dstb)
    zeros16 = jnp.zeros((16,), jnp.float32)

    def zbody(k, carry):
        degp[pl.ds(k * 16, 16)] = zeros16
        return carry

    lax.fori_loop(0, NPAD // 16, zbody, 0)
    ones16 = jnp.ones((16,), jnp.float32)

    def cbody(k, carry):
        idx = dstb[pl.ds(k * 16, 16)]
        plsc.addupdate_scatter(degp, [idx], ones16)
        return carry

    lax.fori_loop(0, EPW // 16, cbody, 0)
    pltpu.sync_copy(degp, out_hbm.at[w])


@functools.partial(
    pl.kernel,
    out_type=jax.ShapeDtypeStruct((NPAD, H), jnp.float32),
    mesh=_mesh,
    scratch_types=[
        pltpu.VMEM((PCH, B), jnp.int32),
        pltpu.VMEM((PCH, B), jnp.int32),
        pltpu.VMEM((B, HH), jnp.float32),
        pltpu.VMEM((B, HH), jnp.float32),
        pltpu.VMEM_SHARED((NPAD, HH), jnp.float32),
        pltpu.SemaphoreType.DMA,
        pltpu.SemaphoreType.DMA,
    ],
    compiler_params=pltpu.CompilerParams(use_tc_tiling_on_sc=False),
)
def _edge_scatter(ys_hbm, src0_hbm, src1_hbm, dst_hbm, zero_hbm, out_hbm,
                  srcb, dstb, g0, g1, acc, sem0, sem1):
    c = lax.axis_index("c")
    s = lax.axis_index("s")
    pltpu.sync_copy(zero_hbm.at[pl.ds(s * NPT, NPT)],
                    acc.at[pl.ds(s * NPT, NPT)])
    plsc.subcore_barrier()

    def start(j, buf, sem):
        pltpu.async_copy(ys_hbm.at[srcb.at[j]], buf, sem)

    def wait(buf, sem):
        pltpu.make_async_copy(ys_hbm.at[srcb.at[0]], buf, sem).wait()

    def scat(j, buf):
        pltpu.sync_copy(buf, acc.at[dstb.at[j]], add=True)

    for p in range(PH):
        base = s * (PCH * PH) + p * PCH

        @pl.when(c == 0)
        def _():
            pltpu.sync_copy(src0_hbm.at[pl.ds(base, PCH)], srcb)

        @pl.when(c == 1)
        def _():
            pltpu.sync_copy(src1_hbm.at[pl.ds(base, PCH)], srcb)

        pltpu.sync_copy(dst_hbm.at[pl.ds(base, PCH)], dstb)
        start(0, g0, sem0)

        def body(i, carry):
            j = 2 * i
            start(j + 1, g1, sem1)
            wait(g0, sem0)
            scat(j, g0)

            @pl.when(j + 2 < PCH)
            def _():
                start(j + 2, g0, sem0)

            wait(g1, sem1)
            scat(j + 1, g1)
            return carry

        lax.fori_loop(0, PCH // 2, body, 0)
    plsc.subcore_barrier()
    pltpu.sync_copy(acc.at[pl.ds(s * NPT, NPT)],
                    out_hbm.at[pl.ds(s * NPT, NPT), pl.ds(c * HH, HH)])


def _dis_body(degp_ref, dis_ref):
    deg = jnp.sum(degp_ref[...], axis=0, keepdims=True) + 1.0
    dis_ref[...] = lax.rsqrt(deg)


def _pre_body(x_ref, wi_ref, bi_ref, w0_ref, dis_ref, h_ref, y_ref):
    h = jnp.maximum(
        jnp.dot(x_ref[...], wi_ref[...], preferred_element_type=jnp.float32)
        + bi_ref[...], 0.0)
    h_ref[...] = h
    y_ref[...] = dis_ref[...] * jnp.dot(
        h, w0_ref[...], preferred_element_type=jnp.float32)


def _ln(x, g, b):
    m = jnp.mean(x, axis=-1, keepdims=True)
    v = jnp.mean((x - m) ** 2, axis=-1, keepdims=True)
    return (x - m) * lax.rsqrt(v + 1e-5) * g + b


def _layer_body(z_ref, y_ref, dis_ref, h_ref, b_ref, g_ref,
                bb_ref, wn_ref, hn_ref, yn_ref):
    dis = dis_ref[...]
    agg = dis * (z_ref[...] + y_ref[...]) + b_ref[...]
    hn = jnp.maximum(_ln(agg, g_ref[...], bb_ref[...]), 0.0) + h_ref[...]
    hn_ref[...] = hn
    yn_ref[...] = dis * jnp.dot(
        hn, wn_ref[...], preferred_element_type=jnp.float32)


def _layer_final_body(z_ref, y_ref, dis_ref, h_ref, b_ref, g_ref, bb_ref,
                      hn_ref):
    agg = dis_ref[...] * (z_ref[...] + y_ref[...]) + b_ref[...]
    hn_ref[...] = jnp.maximum(_ln(agg, g_ref[...], bb_ref[...]), 0.0) + h_ref[...]


def _heads_body(h_ref, nw1_ref, nb1_ref, ng_ref, nbb_ref, nw2_ref, nb2_ref,
                nw3_ref, nb3_ref, gw1_ref, gb1_ref, gg_ref, gbb_ref, gw2_ref,
                gb2_ref, gw3_ref, gb3_ref, nlog_ref, glog_ref, sacc):
    i = pl.program_id(0)
    blk = h_ref[...]

    @pl.when(i == 0)
    def _():
        sacc[0:1, :] = jnp.zeros((1, H), jnp.float32)
        sacc[1:2, :] = jnp.full((1, H), -jnp.inf, jnp.float32)

    sacc[0:1, :] += jnp.sum(blk, axis=0, keepdims=True)
    sacc[1:2, :] = jnp.maximum(sacc[1:2, :], jnp.max(blk, axis=0, keepdims=True))

    t = jnp.dot(blk, nw1_ref[...], preferred_element_type=jnp.float32) + nb1_ref[...]
    t = jnp.maximum(_ln(t, ng_ref[...], nbb_ref[...]), 0.0)
    t = jnp.maximum(
        jnp.dot(t, nw2_ref[...], preferred_element_type=jnp.float32) + nb2_ref[...],
        0.0)
    nlog_ref[...] = jnp.dot(
        t, nw3_ref[...], preferred_element_type=jnp.float32) + nb3_ref[...]

    @pl.when(i == pl.num_programs(0) - 1)
    def _():
        gm = sacc[0:1, :] * (1.0 / N)
        gx = sacc[1:2, :]
        gh = (jnp.dot(gm, gw1_ref[0:H, :], preferred_element_type=jnp.float32)
              + jnp.dot(gx, gw1_ref[H:2 * H, :], preferred_element_type=jnp.float32)
              + gb1_ref[...])
        gh = jnp.maximum(_ln(gh, gg_ref[...], gbb_ref[...]), 0.0)
        gh = jnp.maximum(
            jnp.dot(gh, gw2_ref[...], preferred_element_type=jnp.float32)
            + gb2_ref[...], 0.0)
        gl = jnp.dot(gh, gw3_ref[...], preferred_element_type=jnp.float32) + gb3_ref[...]
        glog_ref[...] = jnp.broadcast_to(gl, (8, H))


def _row_spec(rows):
    return pl.BlockSpec((rows, H), lambda i: (i, 0))


def _full(shape):
    return pl.BlockSpec(shape, lambda i: tuple(0 for _ in shape))


_GRID = N // RB


def _pre_call(x, wi, bi, w0, dis):
    return pl.pallas_call(
        _pre_body,
        grid=(_GRID,),
        in_specs=[_row_spec(RB), _full((H, H)), _full((1, H)), _full((H, H)),
                  pl.BlockSpec((RB, 1), lambda i: (i, 0))],
        out_specs=[_row_spec(RB), _row_spec(RB)],
        out_shape=[jax.ShapeDtypeStruct((N, H), jnp.float32),
                   jax.ShapeDtypeStruct((N, H), jnp.float32)],
    )(x, wi, bi, w0, dis)


def _dis_call(degp):
    return pl.pallas_call(
        _dis_body,
        out_shape=jax.ShapeDtypeStruct((1, NPAD), jnp.float32),
    )(degp)


def _layer_call(z, y, dis, h, b, g, bb, wn):
    return pl.pallas_call(
        _layer_body,
        grid=(_GRID,),
        in_specs=[_row_spec(RB), _row_spec(RB),
                  pl.BlockSpec((RB, 1), lambda i: (i, 0)), _row_spec(RB),
                  _full((1, H)), _full((1, H)), _full((1, H)), _full((H, H))],
        out_specs=[_row_spec(RB), _row_spec(RB)],
        out_shape=[jax.ShapeDtypeStruct((N, H), jnp.float32),
                   jax.ShapeDtypeStruct((N, H), jnp.float32)],
    )(z, y, dis, h, b, g, bb, wn)


def _layer_final_call(z, y, dis, h, b, g, bb):
    return pl.pallas_call(
        _layer_final_body,
        grid=(_GRID,),
        in_specs=[_row_spec(RB), _row_spec(RB),
                  pl.BlockSpec((RB, 1), lambda i: (i, 0)), _row_spec(RB),
                  _full((1, H)), _full((1, H)), _full((1, H))],
        out_specs=_row_spec(RB),
        out_shape=jax.ShapeDtypeStruct((N, H), jnp.float32),
    )(z, y, dis, h, b, g, bb)


def _heads_call(h, nw1, nb1, ng, nbb, nw2, nb2, nw3p, nb3p,
                gw1, gb1, gg, gbb, gw2, gb2, gw3p, gb3p):
    return pl.pallas_call(
        _heads_body,
        grid=(_GRID,),
        in_specs=[_row_spec(RB), _full((H, H)), _full((1, H)), _full((1, H)),
                  _full((1, H)), _full((H, H // 2)), _full((1, H // 2)),
                  _full((H // 2, H)), _full((1, H)), _full((2 * H, H)),
                  _full((1, H)), _full((1, H)), _full((1, H)),
                  _full((H, H // 2)), _full((1, H // 2)), _full((H // 2, H)),
                  _full((1, H))],
        out_specs=[_row_spec(RB), pl.BlockSpec((8, H), lambda i: (0, 0))],
        out_shape=[jax.ShapeDtypeStruct((N, H), jnp.float32),
                   jax.ShapeDtypeStruct((8, H), jnp.float32)],
        scratch_shapes=[pltpu.VMEM((8, H), jnp.float32)],
    )(h, nw1, nb1, ng, nbb, nw2, nb2, nw3p, nb3p,
      gw1, gb1, gg, gbb, gw2, gb2, gw3p, gb3p)


def kernel(x, edge_index, W_in, b_in, conv_W, conv_b, ln_g, ln_b, gW1, gb1,
           gln_g, gln_b, gW2, gb2, gW3, gb3, nW1, nb1, nln_g, nln_b, nW2, nb2,
           nW3, nb3):
    src0_2d = (edge_index[0] * 2).reshape(E // B, B)
    src1_2d = (edge_index[0] * 2 + 1).reshape(E // B, B)
    dst2d = edge_index[1].reshape(E // B, B)

    degp = _deg_count(edge_index[1])
    dis = _dis_call(degp).reshape(NPAD, 1)[:N]

    zeros = jnp.zeros((NPAD, HH), jnp.float32)
    h, y = _pre_call(x, W_in, b_in.reshape(1, H), conv_W[0], dis)
    for i in range(4):
        z = _edge_scatter(y.reshape(2 * N, HH), src0_2d, src1_2d, dst2d,
                          zeros)
        b = conv_b[i].reshape(1, H)
        g = ln_g[i].reshape(1, H)
        bb = ln_b[i].reshape(1, H)
        if i < 3:
            h, y = _layer_call(z, y, dis, h, b, g, bb, conv_W[i + 1])
        else:
            h = _layer_final_call(z, y, dis, h, b, g, bb)

    nw3p = jnp.pad(nW3, ((0, 0), (0, H - 2)))
    nb3p = jnp.pad(nb3, (0, H - 2)).reshape(1, H)
    gw3p = jnp.pad(gW3, ((0, 0), (0, H - 2)))
    gb3p = jnp.pad(gb3, (0, H - 2)).reshape(1, H)
    nlog, glog = _heads_call(
        h, nW1, nb1.reshape(1, H), nln_g.reshape(1, H), nln_b.reshape(1, H),
        nW2, nb2.reshape(1, H // 2), nw3p, nb3p,
        gW1, gb1.reshape(1, H), gln_g.reshape(1, H), gln_b.reshape(1, H),
        gW2, gb2.reshape(1, H // 2), gw3p, gb3p)
    return (glog[0:1, 0:2], nlog[:, 0:2])
